# baseline (speedup 1.0000x reference)
import jax
import jax.numpy as jnp
from jax.experimental import pallas as pl
from jax.experimental.pallas import tpu as pltpu

_B, _C, _H, _W = 64, 768, 24, 24
_HW = _H * _W


def _body(t_ref, x_ref, s_ref, emb_ref, out_ref, mask_ref):
    del t_ref
    m = jax.nn.sigmoid(s_ref[0, 0] * emb_ref[0, 0, :])
    m2 = m[:, None]
    mask_ref[0] = jnp.broadcast_to(m2, (_C, _HW))
    out_ref[0] = x_ref[0] * m2


def kernel(t, x, s, embedding):
    x3 = x.reshape(_B, _C, _HW)
    s2 = s.reshape(1, 1)
    t32 = t.astype(jnp.int32)

    grid_spec = pltpu.PrefetchScalarGridSpec(
        num_scalar_prefetch=1,
        grid=(_B,),
        in_specs=[
            pl.BlockSpec((1, _C, _HW), lambda b, t_ref: (b, 0, 0)),
            pl.BlockSpec((1, 1), lambda b, t_ref: (0, 0)),
            pl.BlockSpec((1, 1, _C), lambda b, t_ref: (t_ref[0], 0, 0)),
        ],
        out_specs=[
            pl.BlockSpec((1, _C, _HW), lambda b, t_ref: (b, 0, 0)),
            pl.BlockSpec((1, _C, _HW), lambda b, t_ref: (b, 0, 0)),
        ],
    )

    out, mask = pl.pallas_call(
        _body,
        grid_spec=grid_spec,
        out_shape=[
            jax.ShapeDtypeStruct((_B, _C, _HW), jnp.float32),
            jax.ShapeDtypeStruct((_B, _C, _HW), jnp.float32),
        ],
    )(t32, x3, s2, embedding.reshape(100, 1, _C))

    return out.reshape(x.shape), mask.reshape(x.shape)


# speedup vs baseline: 1.0284x; 1.0284x over previous
import jax
import jax.numpy as jnp
from jax.experimental import pallas as pl
from jax.experimental.pallas import tpu as pltpu

_B, _C, _H, _W = 64, 768, 24, 24
_HW = _H * _W


_BB = 4


def _body(t_ref, x_ref, s_ref, emb_ref, out_ref, mask_ref):
    del t_ref
    m = jax.nn.sigmoid(s_ref[0, 0] * emb_ref[0, 0, :])
    m2 = m[None, :, None]
    mask_ref[...] = jnp.broadcast_to(m2, (_BB, _C, _HW))
    out_ref[...] = x_ref[...] * m2


def kernel(t, x, s, embedding):
    x3 = x.reshape(_B, _C, _HW)
    s2 = s.reshape(1, 1)
    t32 = t.astype(jnp.int32)

    grid_spec = pltpu.PrefetchScalarGridSpec(
        num_scalar_prefetch=1,
        grid=(_B // _BB,),
        in_specs=[
            pl.BlockSpec((_BB, _C, _HW), lambda b, t_ref: (b, 0, 0)),
            pl.BlockSpec((1, 1), lambda b, t_ref: (0, 0)),
            pl.BlockSpec((1, 1, _C), lambda b, t_ref: (t_ref[0], 0, 0)),
        ],
        out_specs=[
            pl.BlockSpec((_BB, _C, _HW), lambda b, t_ref: (b, 0, 0)),
            pl.BlockSpec((_BB, _C, _HW), lambda b, t_ref: (b, 0, 0)),
        ],
    )

    out, mask = pl.pallas_call(
        _body,
        grid_spec=grid_spec,
        out_shape=[
            jax.ShapeDtypeStruct((_B, _C, _HW), jnp.float32),
            jax.ShapeDtypeStruct((_B, _C, _HW), jnp.float32),
        ],
    )(t32, x3, s2, embedding.reshape(100, 1, _C))

    return out.reshape(x.shape), mask.reshape(x.shape)
